# baseline (speedup 1.0000x reference)
import functools

import jax
import jax.numpy as jnp
from jax import lax
from jax.experimental import pallas as pl
from jax.experimental.pallas import tpu as pltpu
from jax.experimental.pallas import tpu_sc as plsc

_GATHER_DNUMS = lax.GatherDimensionNumbers(
    offset_dims=(), collapsed_slice_dims=(0,), start_index_map=(0,))


def _shuffle(v, idx):
    return lax.gather(v, idx[:, None], _GATHER_DNUMS, slice_sizes=(1,),
                      mode=lax.GatherScatterMode.PROMISE_IN_BOUNDS)


D = 128
L = 16
NC, NS = 2, 16
NW = NC * NS
C = 128
NBUF = 2


def _make_score_kernel(nch):
    EW = nch * C

    mesh = plsc.VectorSubcoreMesh(core_axis_name="c", subcore_axis_name="s")

    @functools.partial(
        pl.kernel,
        mesh=mesh,
        out_type=jax.ShapeDtypeStruct((NW * EW,), jnp.float32),
        scratch_types=[
            pltpu.VMEM((nch, C), jnp.int32),
            pltpu.VMEM((nch, C), jnp.int32),
            pltpu.VMEM((NBUF, C, D), jnp.float32),
            pltpu.VMEM((NBUF, C, D), jnp.float32),
            pltpu.VMEM((EW,), jnp.float32),
            pltpu.SemaphoreType.DMA((NBUF,)),
            pltpu.SemaphoreType.DMA((NBUF,)),
        ],
    )
    def score_k(x_hbm, src_hbm, dst_hbm, out_hbm,
                sidx, didx, srows, drows, outv, ssem, dsem):
        wid = lax.axis_index("s") * NC + lax.axis_index("c")

        pltpu.sync_copy(src_hbm.at[wid], sidx)
        pltpu.sync_copy(dst_hbm.at[wid], didx)

        def issue(g, b):
            pltpu.async_copy(x_hbm.at[sidx.at[g]], srows.at[b], ssem.at[b])
            pltpu.async_copy(x_hbm.at[didx.at[g]], drows.at[b], dsem.at[b])

        def drain(g, b):
            pltpu.make_async_copy(
                x_hbm.at[sidx.at[g]], srows.at[b], ssem.at[b]).wait()
            pltpu.make_async_copy(
                x_hbm.at[didx.at[g]], drows.at[b], dsem.at[b]).wait()

        issue(0, 0)
        lanes = lax.iota(jnp.int32, L)

        def compute(g, b):
            sr = srows.at[b]
            dr = drows.at[b]

            def grp_body(jj, carry2):
                vec = jnp.zeros((L,), jnp.float32)
                for l in range(L):
                    j = jj * L + l
                    acc = jnp.zeros((L,), jnp.float32)
                    for k in range(D // L):
                        a = sr[j, pl.ds(k * L, L)]
                        bb = dr[j, pl.ds(k * L, L)]
                        acc = acc + a * bb
                    for s in (8, 4, 2, 1):
                        acc = acc + _shuffle(acc, lanes ^ s)
                    vec = jnp.where(lanes == l, acc, vec)
                outv[pl.ds(g * C + jj * L, L)] = vec
                return carry2

            lax.fori_loop(0, C // L, grp_body, 0, unroll=False)

        def outer(g2, carry):
            for b in range(NBUF):
                g = g2 * NBUF + b

                @pl.when(g + 1 < nch)
                def _():
                    issue(g + 1, (b + 1) % NBUF)

                drain(g, b)
                compute(g, b)
            return carry

        lax.fori_loop(0, nch // NBUF, outer, 0, unroll=False)

        base0 = pl.multiple_of(wid * EW, 8)
        pltpu.sync_copy(outv, out_hbm.at[pl.ds(base0, EW)])

    return score_k


def kernel(x, edge_index):
    E = edge_index.shape[1]
    ei = edge_index.astype(jnp.int32)
    src = ei[0]
    dst = ei[1]

    step = NW * C * NBUF
    Ep = ((E + step - 1) // step) * step
    if Ep != E:
        src = jnp.pad(src, (0, Ep - E))
        dst = jnp.pad(dst, (0, Ep - E))
    nch = Ep // (NW * C)

    src3 = src.reshape(NW, nch, C)
    dst3 = dst.reshape(NW, nch, C)

    score = _make_score_kernel(nch)(x, src3, dst3)
    return score[:E].reshape(E, 1)
